# Initial kernel scaffold; baseline (speedup 1.0000x reference)
import jax
import jax.numpy as jnp
from jax.experimental import pallas as pl


def kernel(neighs, node_count, table):
    raise NotImplementedError("write your pallas kernel here")


# speedup vs baseline: 4.3858x; 4.3858x over previous
import functools

import jax
import jax.numpy as jnp
from jax import lax
from jax.experimental import pallas as pl
from jax.experimental.pallas import tpu as pltpu
from jax.experimental.pallas import tpu_sc as plsc

N_NODES = 10000
LANES = 16
NUM_CORES = 2
NUM_SUBCORES = 16
NUM_WORKERS = NUM_CORES * NUM_SUBCORES
C_NODES = 16


@functools.partial(jax.jit, static_argnums=(2, 3))
def _mean_agg(neighs, table, k_nb, d_feat):
    n_chunks = N_NODES // C_NODES
    c_rows = C_NODES * k_nb
    dv = d_feat // LANES
    inv_k = jnp.float32(1.0 / k_nb)

    mesh = plsc.VectorSubcoreMesh(
        core_axis_name="c", subcore_axis_name="s", num_cores=NUM_CORES,
        num_subcores=NUM_SUBCORES)

    @functools.partial(
        pl.kernel,
        out_type=jax.ShapeDtypeStruct((N_NODES, d_feat), jnp.float32),
        mesh=mesh,
        scratch_types=[
            pltpu.VMEM((c_rows,), jnp.int32),
            pltpu.VMEM((c_rows, d_feat), jnp.float32),
            pltpu.VMEM((C_NODES, d_feat), jnp.float32),
            pltpu.SemaphoreType.DMA,
        ],
    )
    def k(neighs_hbm, table_hbm, out_hbm, idx_v, rows_v, out_v, sem):
        wid = lax.axis_index("s") * NUM_CORES + lax.axis_index("c")
        my_chunks = (n_chunks - wid + NUM_WORKERS - 1) // NUM_WORKERS

        def chunk_body(i, _):
            chunk = wid + i * NUM_WORKERS
            pltpu.sync_copy(neighs_hbm.at[pl.ds(chunk * c_rows, c_rows)], idx_v)
            pltpu.async_copy(table_hbm.at[idx_v], rows_v, sem).wait()

            def node_body(c, _):
                rbase = c * k_nb

                def red(r, accs):
                    return tuple(
                        accs[d] + rows_v[rbase + r, pl.ds(d * LANES, LANES)]
                        for d in range(dv))

                accs = lax.fori_loop(
                    0, k_nb, red,
                    tuple(jnp.zeros((LANES,), jnp.float32) for _ in range(dv)))
                for d in range(dv):
                    out_v[c, pl.ds(d * LANES, LANES)] = accs[d] * inv_k
                return 0

            lax.fori_loop(0, C_NODES, node_body, 0)
            pltpu.sync_copy(out_v, out_hbm.at[pl.ds(chunk * C_NODES, C_NODES)])
            return 0

        lax.fori_loop(0, my_chunks, chunk_body, 0)

    return k(neighs, table)


def kernel(neighs, node_count, table):
    del node_count
    k_nb = neighs.shape[0] // N_NODES
    return _mean_agg(neighs.astype(jnp.int32), table, k_nb, table.shape[1])


# speedup vs baseline: 5.7275x; 1.3059x over previous
import functools

import jax
import jax.numpy as jnp
from jax import lax
from jax.experimental import pallas as pl
from jax.experimental.pallas import tpu as pltpu
from jax.experimental.pallas import tpu_sc as plsc

N_NODES = 10000
LANES = 16
NUM_CORES = 2
NUM_SUBCORES = 16
NUM_WORKERS = NUM_CORES * NUM_SUBCORES
NPW = 320
C_NODES = 16


@functools.partial(jax.jit, static_argnums=(2, 3))
def _mean_agg(neighs_t, table, k_nb, d_feat):
    inv_k = jnp.float32(1.0 / k_nb)
    n_chunks = NPW // C_NODES

    mesh = plsc.VectorSubcoreMesh(
        core_axis_name="c", subcore_axis_name="s", num_cores=NUM_CORES,
        num_subcores=NUM_SUBCORES)

    @functools.partial(
        pl.kernel,
        out_type=jax.ShapeDtypeStruct((N_NODES, d_feat), jnp.float32),
        mesh=mesh,
        scratch_types=[
            pltpu.VMEM((k_nb * NPW,), jnp.int32),
            pltpu.VMEM((C_NODES, d_feat), jnp.float32),
            pltpu.SemaphoreType.DMA,
        ],
    )
    def k(neighs_hbm, table_hbm, out_hbm, idxt_v, acc_v, sem):
        wid = lax.axis_index("s") * NUM_CORES + lax.axis_index("c")
        start = jnp.minimum(wid * NPW, N_NODES - NPW)
        zeros = jnp.zeros((LANES,), jnp.float32)

        def stage_body(j, _):
            pltpu.sync_copy(neighs_hbm.at[pl.ds(j * N_NODES + start, NPW)],
                            idxt_v.at[pl.ds(j * NPW, NPW)])
            return 0
        lax.fori_loop(0, k_nb, stage_body, 0)

        def chunk_body(cc, _):
            for c in range(C_NODES):
                for d in range(d_feat // LANES):
                    acc_v[c, pl.ds(d * LANES, LANES)] = zeros

            def fire_body(j, _):
                pltpu.async_copy(
                    table_hbm.at[
                        idxt_v.at[pl.ds(j * NPW + cc * C_NODES, C_NODES)]],
                    acc_v, sem, add=True)
                return 0
            lax.fori_loop(0, k_nb, fire_body, 0)

            def drain_body(j, _):
                pltpu.make_async_copy(
                    table_hbm.at[idxt_v.at[pl.ds(0, C_NODES)]],
                    acc_v, sem).wait()
                return 0
            lax.fori_loop(0, k_nb, drain_body, 0)

            for c in range(C_NODES):
                for d in range(d_feat // LANES):
                    sl = pl.ds(d * LANES, LANES)
                    acc_v[c, sl] = acc_v[c, sl] * inv_k
            pltpu.sync_copy(
                acc_v, out_hbm.at[pl.ds(start + cc * C_NODES, C_NODES)])
            return 0

        lax.fori_loop(0, n_chunks, chunk_body, 0)

    return k(neighs_t, table)


def kernel(neighs, node_count, table):
    del node_count
    k_nb = neighs.shape[0] // N_NODES
    neighs_t = neighs.astype(jnp.int32).reshape(N_NODES, k_nb).T.reshape(-1)
    return _mean_agg(neighs_t, table, k_nb, table.shape[1])


# speedup vs baseline: 6.6067x; 1.1535x over previous
import functools

import jax
import jax.numpy as jnp
from jax import lax
from jax.experimental import pallas as pl
from jax.experimental.pallas import tpu as pltpu
from jax.experimental.pallas import tpu_sc as plsc

N_NODES = 10000
LANES = 16
NUM_CORES = 2
NUM_SUBCORES = 16
NUM_WORKERS = NUM_CORES * NUM_SUBCORES
NPW = 320


@functools.partial(jax.jit, static_argnums=(2, 3))
def _mean_agg(neighs_t, table, k_nb, d_feat):
    inv_k = jnp.float32(1.0 / k_nb)

    mesh = plsc.VectorSubcoreMesh(
        core_axis_name="c", subcore_axis_name="s", num_cores=NUM_CORES,
        num_subcores=NUM_SUBCORES)

    @functools.partial(
        pl.kernel,
        out_type=jax.ShapeDtypeStruct((N_NODES, d_feat), jnp.float32),
        mesh=mesh,
        scratch_types=[
            pltpu.VMEM((k_nb * NPW,), jnp.int32),
            pltpu.VMEM((NPW, d_feat), jnp.float32),
            pltpu.SemaphoreType.DMA,
        ],
    )
    def k(neighs_hbm, table_hbm, out_hbm, idxt_v, acc_v, sem):
        wid = lax.axis_index("s") * NUM_CORES + lax.axis_index("c")
        start = jnp.minimum(wid * NPW, N_NODES - NPW)

        def stage_body(j, _):
            pltpu.async_copy(neighs_hbm.at[pl.ds(j * N_NODES + start, NPW)],
                             idxt_v.at[pl.ds(j * NPW, NPW)], sem)
            return 0
        lax.fori_loop(0, k_nb, stage_body, 0)

        zeros = jnp.zeros((LANES,), jnp.float32)

        def zero_body(c, _):
            for d in range(d_feat // LANES):
                acc_v[c, pl.ds(d * LANES, LANES)] = zeros
            return 0
        lax.fori_loop(0, NPW, zero_body, 0)

        def stage_drain(j, _):
            pltpu.make_async_copy(
                neighs_hbm.at[pl.ds(0, NPW)],
                idxt_v.at[pl.ds(0, NPW)], sem).wait()
            return 0
        lax.fori_loop(0, k_nb, stage_drain, 0)

        def fire_body(j, _):
            pltpu.async_copy(table_hbm.at[idxt_v.at[pl.ds(j * NPW, NPW)]],
                             acc_v, sem, add=True)
            return 0
        lax.fori_loop(0, k_nb, fire_body, 0)

        def drain_body(j, _):
            pltpu.make_async_copy(table_hbm.at[idxt_v.at[pl.ds(0, NPW)]],
                                  acc_v, sem).wait()
            return 0
        lax.fori_loop(0, k_nb, drain_body, 0)

        for d in range(d_feat // LANES):
            sl = pl.ds(d * LANES, LANES)

            def scale_body(c, _):
                acc_v[c, sl] = acc_v[c, sl] * inv_k
                return 0
            lax.fori_loop(0, NPW, scale_body, 0)
        pltpu.sync_copy(acc_v, out_hbm.at[pl.ds(start, NPW)])

    return k(neighs_t, table)


def kernel(neighs, node_count, table):
    del node_count
    k_nb = neighs.shape[0] // N_NODES
    neighs_t = neighs.astype(jnp.int32).reshape(N_NODES, k_nb).T.reshape(-1)
    return _mean_agg(neighs_t, table, k_nb, table.shape[1])
